# baseline (speedup 1.0000x reference)
import functools

import jax
import jax.numpy as jnp
from jax import lax
from jax.experimental import pallas as pl
from jax.experimental.pallas import tpu as pltpu
from jax.experimental.pallas import tpu_sc as plsc

B, L, D = 4096, 200, 64
LP = 208
NLANE = 16
CHUNKS = LP // NLANE
HALF = LP // 2
NW = 32
BPW = B // NW
N_CATS_P = 1010
N_PRICE = 11


def _sc_body(ids_hbm, w_hbm, pid_hbm, item_hbm,
             pooled_out, ipos_out,
             ids_v, w_v, rows_v, pooled_v, pid_v, ipos_v, sem):
    wid = lax.axis_index("s") * 2 + lax.axis_index("c")
    base = wid * BPW

    pltpu.sync_copy(ids_hbm.at[pl.ds(base, BPW)], ids_v)
    pltpu.sync_copy(w_hbm.at[pl.ds(base, BPW)], w_v)

    def row_body(r, carry):
        cp0 = pltpu.async_copy(
            item_hbm.at[ids_v.at[r, pl.ds(0, HALF)]],
            rows_v.at[pl.ds(0, HALF)], sem)
        cp1 = pltpu.async_copy(
            item_hbm.at[ids_v.at[r, pl.ds(HALF, HALF)]],
            rows_v.at[pl.ds(HALF, HALF)], sem)

        ws = jnp.zeros((NLANE,), jnp.float32)
        for c in range(CHUNKS):
            sl = pl.ds(c * NLANE, NLANE)
            ids16 = ids_v[r, sl]
            w16 = w_v[r, sl]
            wm = jnp.where(ids16 != 0, w16, 0.0)
            w_v[r, sl] = wm
            ws = ws + wm
        inv = jnp.full((NLANE,), 1.0, jnp.float32) / jnp.maximum(
            jnp.full((NLANE,), jnp.sum(ws), jnp.float32), 1e-8)

        cp0.wait()
        cp1.wait()

        rbase = jnp.full((NLANE,), r, jnp.int32)

        def chunk_body(c, accs):
            a0, a1, a2, a3 = accs
            for k in range(NLANE):
                l = c * NLANE + k
                wb = plsc.load_gather(
                    w_v, [rbase, jnp.full((NLANE,), l, jnp.int32)])
                a0 = a0 + wb * rows_v[l, pl.ds(0, NLANE)]
                a1 = a1 + wb * rows_v[l, pl.ds(NLANE, NLANE)]
                a2 = a2 + wb * rows_v[l, pl.ds(2 * NLANE, NLANE)]
                a3 = a3 + wb * rows_v[l, pl.ds(3 * NLANE, NLANE)]
            return (a0, a1, a2, a3)

        z = jnp.zeros((NLANE,), jnp.float32)
        a0, a1, a2, a3 = lax.fori_loop(0, CHUNKS, chunk_body, (z, z, z, z))
        pooled_v[r, pl.ds(0, NLANE)] = a0 * inv
        pooled_v[r, pl.ds(NLANE, NLANE)] = a1 * inv
        pooled_v[r, pl.ds(2 * NLANE, NLANE)] = a2 * inv
        pooled_v[r, pl.ds(3 * NLANE, NLANE)] = a3 * inv
        return carry

    lax.fori_loop(0, BPW, row_body, 0)
    pltpu.sync_copy(pooled_v, pooled_out.at[pl.ds(base, BPW)])

    pltpu.sync_copy(pid_hbm.at[pl.ds(base, BPW)], pid_v)
    pltpu.async_copy(item_hbm.at[pid_v], ipos_v, sem).wait()
    pltpu.sync_copy(ipos_v, ipos_out.at[pl.ds(base, BPW)])


_sc_pool = pl.kernel(
    _sc_body,
    out_type=(jax.ShapeDtypeStruct((B, D), jnp.float32),
              jax.ShapeDtypeStruct((B, D), jnp.float32)),
    mesh=plsc.VectorSubcoreMesh(core_axis_name="c", subcore_axis_name="s"),
    scratch_types=[
        pltpu.VMEM((BPW, LP), jnp.int32),
        pltpu.VMEM((BPW, LP), jnp.float32),
        pltpu.VMEM((LP, D), jnp.float32),
        pltpu.VMEM((BPW, D), jnp.float32),
        pltpu.VMEM((BPW,), jnp.int32),
        pltpu.VMEM((BPW, D), jnp.float32),
        pltpu.SemaphoreType.DMA,
    ],
    compiler_params=pltpu.CompilerParams(use_tc_tiling_on_sc=False,
                                         needs_layout_passes=False),
)


def _mm(a, b):
    return lax.dot_general(a, b, (((1,), (0,)), ((), ())),
                           precision=lax.Precision.HIGHEST,
                           preferred_element_type=jnp.float32)


def _l2n(x):
    n = jnp.sqrt(jnp.sum(x * x, axis=-1, keepdims=True))
    return x / jnp.maximum(n, 1e-12)


def _tc_body(pooled, ipos, cid, prid, cat_t, price_t,
             W1a, W1b, W1c, b1, W2, b2, W3, b3,
             uW1, ub1, uW2, ub2, uW3, ub3,
             user_out, pos_out):
    h = jnp.maximum(_mm(pooled[...], uW1[...]) + ub1[...], 0.0)
    h = jnp.maximum(_mm(h, uW2[...]) + ub2[...], 0.0)
    u = _mm(h, uW3[...]) + ub3[...]
    user_out[...] = _l2n(u)

    bm = cid.shape[0]
    c_oh = (cid[...] == lax.broadcasted_iota(jnp.int32, (bm, N_CATS_P), 1))
    c_vec = _mm(c_oh.astype(jnp.float32), cat_t[...])
    p_oh = (prid[...] == lax.broadcasted_iota(jnp.int32, (bm, N_PRICE), 1))
    p_vec = _mm(p_oh.astype(jnp.float32), price_t[...])
    x1 = (_mm(ipos[...], W1a[...]) + _mm(c_vec, W1b[...])
          + _mm(p_vec, W1c[...]) + b1[...])
    h = jnp.maximum(x1, 0.0)
    h = jnp.maximum(_mm(h, W2[...]) + b2[...], 0.0)
    v = _mm(h, W3[...]) + b3[...]
    pos_out[...] = _l2n(v)


def _tc_towers(pooled, ipos, cid, prid, cat_t, price_t,
               W1a, W1b, W1c, b1, W2, b2, W3, b3,
               uW1, ub1, uW2, ub2, uW3, ub3):
    BM = 512
    grid = (B // BM,)

    def row_spec(w):
        return pl.BlockSpec((BM, w), lambda i: (i, 0))

    def full_spec(shape):
        return pl.BlockSpec(shape, lambda i: (0,) * len(shape))

    in_specs = [
        row_spec(D), row_spec(D), row_spec(1), row_spec(1),
        full_spec(cat_t.shape), full_spec(price_t.shape),
        full_spec(W1a.shape), full_spec(W1b.shape), full_spec(W1c.shape),
        full_spec(b1.shape), full_spec(W2.shape), full_spec(b2.shape),
        full_spec(W3.shape), full_spec(b3.shape),
        full_spec(uW1.shape), full_spec(ub1.shape), full_spec(uW2.shape),
        full_spec(ub2.shape), full_spec(uW3.shape), full_spec(ub3.shape),
    ]
    out_specs = [row_spec(D), row_spec(D)]
    return pl.pallas_call(
        _tc_body,
        grid=grid,
        in_specs=in_specs,
        out_specs=out_specs,
        out_shape=[jax.ShapeDtypeStruct((B, D), jnp.float32),
                   jax.ShapeDtypeStruct((B, D), jnp.float32)],
    )(pooled, ipos, cid, prid, cat_t, price_t,
      W1a, W1b, W1c, b1, W2, b2, W3, b3,
      uW1, ub1, uW2, ub2, uW3, ub3)


def kernel(history_ids, history_weights, pos_item_ids, pos_cat_ids, pos_price,
           item_emb, cat_emb, price_emb,
           it_W1, it_b1, it_W2, it_b2, it_W3, it_b3,
           us_W1, us_b1, us_W2, us_b2, us_W3, us_b3):
    ids = jnp.pad(history_ids.astype(jnp.int32), ((0, 0), (0, LP - L)))
    w = jnp.pad(history_weights, ((0, 0), (0, LP - L)))
    pid = pos_item_ids.astype(jnp.int32)

    pooled, ipos = _sc_pool(ids, w, pid, item_emb)

    cid = pos_cat_ids.astype(jnp.int32).reshape(B, 1)
    prid = pos_price.astype(jnp.int32).reshape(B, 1)
    W1a, W1b, W1c = it_W1[:D], it_W1[D:D + 50], it_W1[D + 50:]
    user_emb, pos_emb = _tc_towers(
        pooled, ipos, cid, prid, cat_emb, price_emb,
        W1a, W1b, W1c, it_b1.reshape(1, -1), it_W2, it_b2.reshape(1, -1),
        it_W3, it_b3.reshape(1, -1),
        us_W1, us_b1.reshape(1, -1), us_W2, us_b2.reshape(1, -1),
        us_W3, us_b3.reshape(1, -1))
    return (user_emb, pos_emb)


# speedup vs baseline: 1.7574x; 1.7574x over previous
import functools

import jax
import jax.numpy as jnp
from jax import lax
from jax.experimental import pallas as pl
from jax.experimental.pallas import tpu as pltpu
from jax.experimental.pallas import tpu_sc as plsc

B, L, D = 4096, 200, 64
NLANE = 16
NFULL = 12
TAIL_OFF = L - NLANE
H0, H1 = 104, 96
NW = 32
BPW = B // NW
N_CATS_P = 1010
N_PRICE = 11


def _fire(item_hbm, ids_v, r, rows_buf, sem):
    cp0 = pltpu.async_copy(
        item_hbm.at[ids_v.at[r, pl.ds(0, H0)]], rows_buf.at[pl.ds(0, H0)], sem)
    cp1 = pltpu.async_copy(
        item_hbm.at[ids_v.at[r, pl.ds(H0, H1)]], rows_buf.at[pl.ds(H0, H1)],
        sem)
    return cp0, cp1


def _wait(item_hbm, ids_v, r, rows_buf, sem):
    pltpu.make_async_copy(
        item_hbm.at[ids_v.at[r, pl.ds(0, H0)]], rows_buf.at[pl.ds(0, H0)],
        sem).wait()
    pltpu.make_async_copy(
        item_hbm.at[ids_v.at[r, pl.ds(H0, H1)]], rows_buf.at[pl.ds(H0, H1)],
        sem).wait()


def _compute_row(ids_v, w_v, pooled_v, rows_buf, r):
    lane = lax.iota(jnp.int32, NLANE)
    z = jnp.zeros((NLANE,), jnp.float32)

    def chunk_body(c, carry):
        a0, a1, a2, a3, ws = carry
        off = c * NLANE
        ich = ids_v[r, pl.ds(off, NLANE)]
        wch = w_v[r, pl.ds(off, NLANE)]
        wm = jnp.where(ich != 0, wch, 0.0)
        ws = ws + wm
        for k in range(NLANE):
            wb = jnp.full((NLANE,), wm[k], jnp.float32)
            base = off + k
            a0 = a0 + wb * rows_buf[base, pl.ds(0, NLANE)]
            a1 = a1 + wb * rows_buf[base, pl.ds(NLANE, NLANE)]
            a2 = a2 + wb * rows_buf[base, pl.ds(2 * NLANE, NLANE)]
            a3 = a3 + wb * rows_buf[base, pl.ds(3 * NLANE, NLANE)]
        return (a0, a1, a2, a3, ws)

    a0, a1, a2, a3, ws = lax.fori_loop(0, NFULL, chunk_body, (z, z, z, z, z))

    ich = ids_v[r, pl.ds(TAIL_OFF, NLANE)]
    wch = w_v[r, pl.ds(TAIL_OFF, NLANE)]
    wm = jnp.where((lane >= 8) & (ich != 0), wch, 0.0)
    ws = ws + wm
    for k in range(8, NLANE):
        wb = jnp.full((NLANE,), wm[k], jnp.float32)
        base = TAIL_OFF + k
        a0 = a0 + wb * rows_buf[base, pl.ds(0, NLANE)]
        a1 = a1 + wb * rows_buf[base, pl.ds(NLANE, NLANE)]
        a2 = a2 + wb * rows_buf[base, pl.ds(2 * NLANE, NLANE)]
        a3 = a3 + wb * rows_buf[base, pl.ds(3 * NLANE, NLANE)]

    inv = jnp.full((NLANE,), 1.0, jnp.float32) / jnp.maximum(
        jnp.full((NLANE,), jnp.sum(ws), jnp.float32), 1e-8)
    pooled_v[r, pl.ds(0, NLANE)] = a0 * inv
    pooled_v[r, pl.ds(NLANE, NLANE)] = a1 * inv
    pooled_v[r, pl.ds(2 * NLANE, NLANE)] = a2 * inv
    pooled_v[r, pl.ds(3 * NLANE, NLANE)] = a3 * inv


def _sc_body(ids_hbm, w_hbm, pid_hbm, item_hbm,
             pooled_out, ipos_out,
             ids_v, w_v, rows_a, rows_b, pooled_v, pid_v, ipos_v,
             sem_a, sem_b, sem_p):
    wid = lax.axis_index("s") * 2 + lax.axis_index("c")
    base = wid * BPW

    pltpu.sync_copy(ids_hbm.at[pl.ds(base, BPW)], ids_v)
    pltpu.sync_copy(w_hbm.at[pl.ds(base, BPW)], w_v)
    pltpu.sync_copy(pid_hbm.at[pl.ds(base, BPW)], pid_v)
    pltpu.async_copy(item_hbm.at[pid_v], ipos_v, sem_p)

    _fire(item_hbm, ids_v, 0, rows_a, sem_a)

    def pair_body(i, carry):
        r0 = 2 * i
        _fire(item_hbm, ids_v, r0 + 1, rows_b, sem_b)
        _wait(item_hbm, ids_v, r0, rows_a, sem_a)
        _compute_row(ids_v, w_v, pooled_v, rows_a, r0)

        @pl.when(i < BPW // 2 - 1)
        def _():
            _fire(item_hbm, ids_v, r0 + 2, rows_a, sem_a)

        _wait(item_hbm, ids_v, r0 + 1, rows_b, sem_b)
        _compute_row(ids_v, w_v, pooled_v, rows_b, r0 + 1)
        return carry

    lax.fori_loop(0, BPW // 2, pair_body, 0)

    pltpu.sync_copy(pooled_v, pooled_out.at[pl.ds(base, BPW)])
    pltpu.make_async_copy(item_hbm.at[pid_v], ipos_v, sem_p).wait()
    pltpu.sync_copy(ipos_v, ipos_out.at[pl.ds(base, BPW)])


_sc_pool = pl.kernel(
    _sc_body,
    out_type=(jax.ShapeDtypeStruct((B, D), jnp.float32),
              jax.ShapeDtypeStruct((B, D), jnp.float32)),
    mesh=plsc.VectorSubcoreMesh(core_axis_name="c", subcore_axis_name="s"),
    scratch_types=[
        pltpu.VMEM((BPW, L), jnp.int32),
        pltpu.VMEM((BPW, L), jnp.float32),
        pltpu.VMEM((L, D), jnp.float32),
        pltpu.VMEM((L, D), jnp.float32),
        pltpu.VMEM((BPW, D), jnp.float32),
        pltpu.VMEM((BPW,), jnp.int32),
        pltpu.VMEM((BPW, D), jnp.float32),
        pltpu.SemaphoreType.DMA,
        pltpu.SemaphoreType.DMA,
        pltpu.SemaphoreType.DMA,
    ],
    compiler_params=pltpu.CompilerParams(use_tc_tiling_on_sc=False,
                                         needs_layout_passes=False),
)


def _mm(a, b):
    return lax.dot_general(a, b, (((1,), (0,)), ((), ())),
                           precision=lax.Precision.HIGHEST,
                           preferred_element_type=jnp.float32)


def _l2n(x):
    n = jnp.sqrt(jnp.sum(x * x, axis=-1, keepdims=True))
    return x / jnp.maximum(n, 1e-12)


def _tc_body(pooled, ipos, cid, prid, cat_t, price_t,
             W1a, W1b, W1c, b1, W2, b2, W3, b3,
             uW1, ub1, uW2, ub2, uW3, ub3,
             user_out, pos_out):
    h = jnp.maximum(_mm(pooled[...], uW1[...]) + ub1[...], 0.0)
    h = jnp.maximum(_mm(h, uW2[...]) + ub2[...], 0.0)
    u = _mm(h, uW3[...]) + ub3[...]
    user_out[...] = _l2n(u)

    bm = cid.shape[0]
    c_oh = (cid[...] == lax.broadcasted_iota(jnp.int32, (bm, N_CATS_P), 1))
    c_vec = _mm(c_oh.astype(jnp.float32), cat_t[...])
    p_oh = (prid[...] == lax.broadcasted_iota(jnp.int32, (bm, N_PRICE), 1))
    p_vec = _mm(p_oh.astype(jnp.float32), price_t[...])
    x1 = (_mm(ipos[...], W1a[...]) + _mm(c_vec, W1b[...])
          + _mm(p_vec, W1c[...]) + b1[...])
    h = jnp.maximum(x1, 0.0)
    h = jnp.maximum(_mm(h, W2[...]) + b2[...], 0.0)
    v = _mm(h, W3[...]) + b3[...]
    pos_out[...] = _l2n(v)


def _tc_towers(pooled, ipos, cid, prid, cat_t, price_t,
               W1a, W1b, W1c, b1, W2, b2, W3, b3,
               uW1, ub1, uW2, ub2, uW3, ub3):
    BM = 512
    grid = (B // BM,)

    def row_spec(w):
        return pl.BlockSpec((BM, w), lambda i: (i, 0))

    def full_spec(shape):
        return pl.BlockSpec(shape, lambda i: (0,) * len(shape))

    in_specs = [
        row_spec(D), row_spec(D), row_spec(1), row_spec(1),
        full_spec(cat_t.shape), full_spec(price_t.shape),
        full_spec(W1a.shape), full_spec(W1b.shape), full_spec(W1c.shape),
        full_spec(b1.shape), full_spec(W2.shape), full_spec(b2.shape),
        full_spec(W3.shape), full_spec(b3.shape),
        full_spec(uW1.shape), full_spec(ub1.shape), full_spec(uW2.shape),
        full_spec(ub2.shape), full_spec(uW3.shape), full_spec(ub3.shape),
    ]
    out_specs = [row_spec(D), row_spec(D)]
    return pl.pallas_call(
        _tc_body,
        grid=grid,
        in_specs=in_specs,
        out_specs=out_specs,
        out_shape=[jax.ShapeDtypeStruct((B, D), jnp.float32),
                   jax.ShapeDtypeStruct((B, D), jnp.float32)],
    )(pooled, ipos, cid, prid, cat_t, price_t,
      W1a, W1b, W1c, b1, W2, b2, W3, b3,
      uW1, ub1, uW2, ub2, uW3, ub3)


def kernel(history_ids, history_weights, pos_item_ids, pos_cat_ids, pos_price,
           item_emb, cat_emb, price_emb,
           it_W1, it_b1, it_W2, it_b2, it_W3, it_b3,
           us_W1, us_b1, us_W2, us_b2, us_W3, us_b3):
    ids = history_ids.astype(jnp.int32)
    pid = pos_item_ids.astype(jnp.int32)

    pooled, ipos = _sc_pool(ids, history_weights, pid, item_emb)

    cid = pos_cat_ids.astype(jnp.int32).reshape(B, 1)
    prid = pos_price.astype(jnp.int32).reshape(B, 1)
    W1a, W1b, W1c = it_W1[:D], it_W1[D:D + 50], it_W1[D + 50:]
    user_emb, pos_emb = _tc_towers(
        pooled, ipos, cid, prid, cat_emb, price_emb,
        W1a, W1b, W1c, it_b1.reshape(1, -1), it_W2, it_b2.reshape(1, -1),
        it_W3, it_b3.reshape(1, -1),
        us_W1, us_b1.reshape(1, -1), us_W2, us_b2.reshape(1, -1),
        us_W3, us_b3.reshape(1, -1))
    return (user_emb, pos_emb)
